# baseline (speedup 1.0000x reference)
import functools

import jax
import jax.numpy as jnp
from jax.experimental import pallas as pl
from jax.experimental.pallas import tpu as pltpu

KNB = 32
TAU2 = 4.0
NQ = 1024
NE = 100000
ED = 32
NT = 32


QB = 128


def _combine_body(sq_ref, gev_ref, gce_ref, lbe_ref, lbc_ref, out_ref):
    sq = sq_ref[...]
    w = jnp.exp(-sq) * (sq <= TAU2).astype(jnp.float32)
    ia = jax.lax.broadcasted_iota(jnp.int32, (NT, NT), 0)
    ib = jax.lax.broadcasted_iota(jnp.int32, (NT, NT), 1)
    ut = (ia >= ib).astype(jnp.float32)
    ev = jnp.exp(gev_ref[...])
    ar = ev + jnp.exp(gce_ref[...])
    risk = jnp.dot(ar, ut, preferred_element_type=jnp.float32,
                   precision=jax.lax.Precision.HIGHEST)
    w3 = w[:, :, None]
    nm = jnp.sum(ev.reshape(QB, KNB, NT) * w3, axis=1)
    dn = jnp.sum(risk.reshape(QB, KNB, NT) * w3, axis=1)
    bev = jnp.exp(lbe_ref[...])
    bar = jnp.dot(bev + jnp.exp(lbc_ref[...]), ut,
                  preferred_element_type=jnp.float32,
                  precision=jax.lax.Precision.HIGHEST)
    numer = nm + bev
    denom = dn + bar + 1e-12
    out_ref[...] = jnp.clip(numer / denom, 1e-12, 1.0 - 1e-12)


@functools.partial(jax.jit)
def _combine(sq, gev, gce, lbe, lbc):
    return pl.pallas_call(
        _combine_body,
        grid=(NQ // QB,),
        in_specs=[
            pl.BlockSpec((QB, KNB), lambda i: (i, 0)),
            pl.BlockSpec((QB * KNB, NT), lambda i: (i, 0)),
            pl.BlockSpec((QB * KNB, NT), lambda i: (i, 0)),
            pl.BlockSpec((1, NT), lambda i: (0, 0)),
            pl.BlockSpec((1, NT), lambda i: (0, 0)),
        ],
        out_specs=pl.BlockSpec((QB, NT), lambda i: (i, 0)),
        out_shape=jax.ShapeDtypeStruct((NQ, NT), jnp.float32),
    )(sq, gev, gce, lbe, lbc)


def _sc_gather_body(ev_hbm, ce_hbm, idx_hbm, oev_hbm, oce_hbm,
                    idx_v, rows_ev, rows_ce, sem):
    wid = jax.lax.axis_index("s") * 2 + jax.lax.axis_index("c")
    bpw = idx_v.shape[0]
    base = wid * bpw
    pltpu.sync_copy(idx_hbm.at[pl.ds(base, bpw)], idx_v)
    pltpu.async_copy(ev_hbm.at[idx_v], rows_ev, sem).wait()
    pltpu.sync_copy(rows_ev, oev_hbm.at[pl.ds(base, bpw)])
    pltpu.async_copy(ce_hbm.at[idx_v], rows_ce, sem).wait()
    pltpu.sync_copy(rows_ce, oce_hbm.at[pl.ds(base, bpw)])


def _sc_gather(ev, ce, idx):
    from jax.experimental.pallas import tpu_sc as plsc
    b = idx.shape[0]
    nt = ev.shape[1]
    nw = 32
    bpw = b // nw
    mesh = plsc.VectorSubcoreMesh(core_axis_name="c", subcore_axis_name="s")
    f = pl.kernel(
        _sc_gather_body,
        mesh=mesh,
        compiler_params=pltpu.CompilerParams(use_tc_tiling_on_sc=False),
        out_type=[
            jax.ShapeDtypeStruct((b, nt), jnp.float32),
            jax.ShapeDtypeStruct((b, nt), jnp.float32),
        ],
        scratch_types=[
            pltpu.VMEM((bpw,), jnp.int32),
            pltpu.VMEM((bpw, nt), jnp.float32),
            pltpu.VMEM((bpw, nt), jnp.float32),
            pltpu.SemaphoreType.DMA,
        ],
    )
    return f(ev, ce, idx)


_F32_INF = float("inf")
_I32_BIG = 2**31 - 1


def _rowmin_arg(x, idx):
    m = jnp.min(x, axis=1, keepdims=True)
    cand = jnp.where(x == m, idx, _I32_BIG)
    return m, jnp.min(cand, axis=1, keepdims=True)


def _knn_body(nb, eb, qm2_ref, e_ref, e2_ref, tv_ref, ti_ref,
              bv_ref, bi_ref, lane_ref):
    i = pl.program_id(0)
    j = jax.lax.rem(i, nb)
    nq = qm2_ref.shape[0]
    qm2 = qm2_ref[...]
    e = e_ref[...]
    q2 = 0.25 * jnp.sum(qm2 * qm2, axis=1, keepdims=True)
    mm = jax.lax.dot_general(qm2, e, (((1,), (1,)), ((), ())),
                             preferred_element_type=jnp.float32)
    d2 = jnp.maximum((q2 + e2_ref[...]) + mm, 0.0)

    @pl.when(i == 0)
    def _init():
        bv_ref[...] = jnp.full((nq, eb), _F32_INF, jnp.float32)
        bi_ref[...] = jnp.zeros((nq, eb), jnp.int32)
        lane_ref[...] = jax.lax.broadcasted_iota(jnp.int32, (nq, eb), 1)

    @pl.when(i < nb)
    def _phase1():
        bv = bv_ref[...]
        sel = d2 < bv
        bv_ref[...] = jnp.where(sel, d2, bv)
        bi_ref[...] = jnp.where(sel, j, bi_ref[...])

    @pl.when(i == nb)
    def _extract():
        lane = lane_ref[...]
        bv = bv_ref[...]
        gb = bi_ref[...] * eb + lane
        vals, idxs = [], []
        for _ in range(KNB):
            m, am = _rowmin_arg(bv, lane)
            vals.append(m)
            hit = lane == am
            idxs.append(jnp.min(jnp.where(hit, gb, _I32_BIG), axis=1,
                                keepdims=True))
            bv = jnp.where(hit, _F32_INF, bv)
        tv_ref[...] = jnp.concatenate(vals, axis=1)
        ti_ref[...] = jnp.concatenate(idxs, axis=1)

    @pl.when(i >= nb)
    def _phase2():
        lane = lane_ref[...]
        d2m = jnp.where(bi_ref[...] == j, _F32_INF, d2)
        tv = tv_ref[...]
        ti = ti_ref[...]
        t = jnp.max(tv, axis=1, keepdims=True)
        lane32 = jax.lax.broadcasted_iota(jnp.int32, (nq, KNB), 1)
        m0 = jnp.min(d2m, axis=1, keepdims=True)

        def cond(c):
            v_last, l_last, m, tv, ti, t = c
            return jnp.any(m < t)

        def body(c):
            v_last, l_last, m, tv, ti, t = c
            act = (d2m > v_last) | ((d2m == v_last) & (lane > l_last))
            lm = jnp.min(jnp.where(act & (d2m == m), lane, _I32_BIG),
                         axis=1, keepdims=True)
            ins = m < t
            pos = jnp.min(jnp.where(tv == t, lane32, _I32_BIG), axis=1,
                          keepdims=True)
            hit = ins & (lane32 == pos)
            tv = jnp.where(hit, m, tv)
            ti = jnp.where(hit, j * eb + lm, ti)
            t = jnp.max(tv, axis=1, keepdims=True)
            act2 = (d2m > m) | ((d2m == m) & (lane > lm))
            m2 = jnp.min(jnp.where(act2, d2m, _F32_INF), axis=1, keepdims=True)
            return m, lm, m2, tv, ti, t

        init = (jnp.full((nq, 1), -_F32_INF, jnp.float32),
                jnp.full((nq, 1), -1, jnp.int32), m0, tv, ti, t)
        _, _, _, tv, ti, _ = jax.lax.while_loop(cond, body, init)
        tv_ref[...] = tv
        ti_ref[...] = ti


def _knn(q, emb, eb=1024):
    nq, ed = q.shape
    ne = emb.shape[0]
    nb = (ne + eb - 1) // eb
    pad = nb * eb - ne
    e2 = jnp.sum(emb * emb, axis=1)
    if pad:
        emb = jnp.concatenate([emb, jnp.zeros((pad, ed), jnp.float32)], axis=0)
        e2 = jnp.concatenate([e2, jnp.full((pad,), 1e9, jnp.float32)], axis=0)
    e2 = e2.reshape(1, nb * eb)
    body = functools.partial(_knn_body, nb, eb)
    return pl.pallas_call(
        body,
        grid=(2 * nb,),
        in_specs=[
            pl.BlockSpec((nq, ed), lambda i: (0, 0)),
            pl.BlockSpec((eb, ed), lambda i: (jax.lax.rem(i, nb), 0)),
            pl.BlockSpec((1, eb), lambda i: (0, jax.lax.rem(i, nb))),
        ],
        out_specs=[
            pl.BlockSpec((nq, KNB), lambda i: (0, 0)),
            pl.BlockSpec((nq, KNB), lambda i: (0, 0)),
        ],
        out_shape=[
            jax.ShapeDtypeStruct((nq, KNB), jnp.float32),
            jax.ShapeDtypeStruct((nq, KNB), jnp.int32),
        ],
        scratch_shapes=[
            pltpu.VMEM((nq, eb), jnp.float32),
            pltpu.VMEM((nq, eb), jnp.int32),
            pltpu.VMEM((nq, eb), jnp.int32),
        ],
    )(-2.0 * q, emb, e2)


def kernel(input, exemplar_embeddings, log_exemplar_event_counts,
           log_exemplar_censor_counts, log_baseline_event_counts,
           log_baseline_censor_counts):
    sq, labels = _knn(input, exemplar_embeddings)
    idx = labels.reshape(-1)
    gev, gce = _sc_gather(log_exemplar_event_counts,
                          log_exemplar_censor_counts, idx)
    lbe = log_baseline_event_counts.reshape(1, NT)
    lbc = log_baseline_censor_counts.reshape(1, NT)
    return _combine(sq, gev, gce, lbe, lbc)


# speedup vs baseline: 1.0754x; 1.0754x over previous
import functools

import jax
import jax.numpy as jnp
from jax.experimental import pallas as pl
from jax.experimental.pallas import tpu as pltpu

KNB = 32
TAU2 = 4.0
NQ = 1024
NE = 100000
ED = 32
NT = 32


QB = 128


def _combine_body(sq_ref, gev_ref, gce_ref, lbe_ref, lbc_ref, out_ref):
    sq = sq_ref[...]
    w = jnp.exp(-sq) * (sq <= TAU2).astype(jnp.float32)
    ia = jax.lax.broadcasted_iota(jnp.int32, (NT, NT), 0)
    ib = jax.lax.broadcasted_iota(jnp.int32, (NT, NT), 1)
    ut = (ia >= ib).astype(jnp.float32)
    ev = jnp.exp(gev_ref[...])
    ar = ev + jnp.exp(gce_ref[...])
    risk = jnp.dot(ar, ut, preferred_element_type=jnp.float32,
                   precision=jax.lax.Precision.HIGHEST)
    w3 = w[:, :, None]
    nm = jnp.sum(ev.reshape(QB, KNB, NT) * w3, axis=1)
    dn = jnp.sum(risk.reshape(QB, KNB, NT) * w3, axis=1)
    bev = jnp.exp(lbe_ref[...])
    bar = jnp.dot(bev + jnp.exp(lbc_ref[...]), ut,
                  preferred_element_type=jnp.float32,
                  precision=jax.lax.Precision.HIGHEST)
    numer = nm + bev
    denom = dn + bar + 1e-12
    out_ref[...] = jnp.clip(numer / denom, 1e-12, 1.0 - 1e-12)


@functools.partial(jax.jit)
def _combine(sq, gev, gce, lbe, lbc):
    return pl.pallas_call(
        _combine_body,
        grid=(NQ // QB,),
        in_specs=[
            pl.BlockSpec((QB, KNB), lambda i: (i, 0)),
            pl.BlockSpec((QB * KNB, NT), lambda i: (i, 0)),
            pl.BlockSpec((QB * KNB, NT), lambda i: (i, 0)),
            pl.BlockSpec((1, NT), lambda i: (0, 0)),
            pl.BlockSpec((1, NT), lambda i: (0, 0)),
        ],
        out_specs=pl.BlockSpec((QB, NT), lambda i: (i, 0)),
        out_shape=jax.ShapeDtypeStruct((NQ, NT), jnp.float32),
    )(sq, gev, gce, lbe, lbc)


def _sc_gather_body(ev_hbm, ce_hbm, idx_hbm, oev_hbm, oce_hbm,
                    idx_v, rows_ev, rows_ce, sem):
    wid = jax.lax.axis_index("s") * 2 + jax.lax.axis_index("c")
    bpw = idx_v.shape[0]
    base = wid * bpw
    pltpu.sync_copy(idx_hbm.at[pl.ds(base, bpw)], idx_v)
    pltpu.async_copy(ev_hbm.at[idx_v], rows_ev, sem).wait()
    pltpu.sync_copy(rows_ev, oev_hbm.at[pl.ds(base, bpw)])
    pltpu.async_copy(ce_hbm.at[idx_v], rows_ce, sem).wait()
    pltpu.sync_copy(rows_ce, oce_hbm.at[pl.ds(base, bpw)])


def _sc_gather(ev, ce, idx):
    from jax.experimental.pallas import tpu_sc as plsc
    b = idx.shape[0]
    nt = ev.shape[1]
    nw = 32
    bpw = b // nw
    mesh = plsc.VectorSubcoreMesh(core_axis_name="c", subcore_axis_name="s")
    f = pl.kernel(
        _sc_gather_body,
        mesh=mesh,
        compiler_params=pltpu.CompilerParams(use_tc_tiling_on_sc=False),
        out_type=[
            jax.ShapeDtypeStruct((b, nt), jnp.float32),
            jax.ShapeDtypeStruct((b, nt), jnp.float32),
        ],
        scratch_types=[
            pltpu.VMEM((bpw,), jnp.int32),
            pltpu.VMEM((bpw, nt), jnp.float32),
            pltpu.VMEM((bpw, nt), jnp.float32),
            pltpu.SemaphoreType.DMA,
        ],
    )
    return f(ev, ce, idx)


_F32_INF = float("inf")
_I32_BIG = 2**31 - 1


def _rowmin_arg(x, idx):
    m = jnp.min(x, axis=1, keepdims=True)
    cand = jnp.where(x == m, idx, _I32_BIG)
    return m, jnp.min(cand, axis=1, keepdims=True)


def _knn_body(nb, eb, qm2_ref, e_ref, e2_ref, tv_ref, ti_ref,
              bv_ref, bi_ref):
    i = pl.program_id(0)
    j = jax.lax.rem(i, nb)
    nq = qm2_ref.shape[0]
    qm2 = qm2_ref[...]
    e = e_ref[...]
    q2 = 0.25 * jnp.sum(qm2 * qm2, axis=1, keepdims=True)
    mm = jax.lax.dot_general(qm2, e, (((1,), (1,)), ((), ())),
                             preferred_element_type=jnp.float32)
    d2 = jnp.maximum((q2 + e2_ref[...]) + mm, 0.0)

    @pl.when(i == 0)
    def _init():
        bv_ref[...] = jnp.full((nq, eb), _F32_INF, jnp.float32)
        bi_ref[...] = jnp.zeros((nq, eb), jnp.int32)

    @pl.when(i < nb)
    def _phase1():
        bv = bv_ref[...]
        sel = d2 < bv
        bv_ref[...] = jnp.where(sel, d2, bv)
        bi_ref[...] = jnp.where(sel, j, bi_ref[...])

    @pl.when(i == nb)
    def _extract():
        lane = jax.lax.broadcasted_iota(jnp.int32, (nq, eb), 1)
        bv = bv_ref[...]
        gb = bi_ref[...] * eb + lane
        vals, idxs = [], []
        for _ in range(KNB):
            m, am = _rowmin_arg(bv, lane)
            vals.append(m)
            hit = lane == am
            idxs.append(jnp.min(jnp.where(hit, gb, _I32_BIG), axis=1,
                                keepdims=True))
            bv = jnp.where(hit, _F32_INF, bv)
        tv_ref[...] = jnp.concatenate(vals, axis=1)
        ti_ref[...] = jnp.concatenate(idxs, axis=1)

    @pl.when(i >= nb)
    def _phase2():
        lane = jax.lax.broadcasted_iota(jnp.int32, (nq, eb), 1)
        d2m = jnp.where(bi_ref[...] == j, _F32_INF, d2)
        tv = tv_ref[...]
        ti = ti_ref[...]
        t = jnp.max(tv, axis=1, keepdims=True)
        lane32 = jax.lax.broadcasted_iota(jnp.int32, (nq, KNB), 1)
        m0 = jnp.min(d2m, axis=1, keepdims=True)

        def cond(c):
            v_last, l_last, m, tv, ti, t = c
            return jnp.any(m < t)

        def body(c):
            v_last, l_last, m, tv, ti, t = c
            act = (d2m > v_last) | ((d2m == v_last) & (lane > l_last))
            lm = jnp.min(jnp.where(act & (d2m == m), lane, _I32_BIG),
                         axis=1, keepdims=True)
            ins = m < t
            pos = jnp.min(jnp.where(tv == t, lane32, _I32_BIG), axis=1,
                          keepdims=True)
            hit = ins & (lane32 == pos)
            tv = jnp.where(hit, m, tv)
            ti = jnp.where(hit, j * eb + lm, ti)
            t = jnp.max(tv, axis=1, keepdims=True)
            act2 = (d2m > m) | ((d2m == m) & (lane > lm))
            m2 = jnp.min(jnp.where(act2, d2m, _F32_INF), axis=1, keepdims=True)
            return m, lm, m2, tv, ti, t

        init = (jnp.full((nq, 1), -_F32_INF, jnp.float32),
                jnp.full((nq, 1), -1, jnp.int32), m0, tv, ti, t)
        _, _, _, tv, ti, _ = jax.lax.while_loop(cond, body, init)
        tv_ref[...] = tv
        ti_ref[...] = ti


def _knn(q, emb, eb=1024):
    nq, ed = q.shape
    ne = emb.shape[0]
    nb = (ne + eb - 1) // eb
    pad = nb * eb - ne
    e2 = jnp.sum(emb * emb, axis=1)
    if pad:
        emb = jnp.concatenate([emb, jnp.zeros((pad, ed), jnp.float32)], axis=0)
        e2 = jnp.concatenate([e2, jnp.full((pad,), 1e9, jnp.float32)], axis=0)
    e2 = e2.reshape(1, nb * eb)
    body = functools.partial(_knn_body, nb, eb)
    return pl.pallas_call(
        body,
        grid=(2 * nb,),
        in_specs=[
            pl.BlockSpec((nq, ed), lambda i: (0, 0)),
            pl.BlockSpec((eb, ed), lambda i: (jax.lax.rem(i, nb), 0)),
            pl.BlockSpec((1, eb), lambda i: (0, jax.lax.rem(i, nb))),
        ],
        out_specs=[
            pl.BlockSpec((nq, KNB), lambda i: (0, 0)),
            pl.BlockSpec((nq, KNB), lambda i: (0, 0)),
        ],
        out_shape=[
            jax.ShapeDtypeStruct((nq, KNB), jnp.float32),
            jax.ShapeDtypeStruct((nq, KNB), jnp.int32),
        ],
        scratch_shapes=[
            pltpu.VMEM((nq, eb), jnp.float32),
            pltpu.VMEM((nq, eb), jnp.int32),
        ],
    )(-2.0 * q, emb, e2)


def kernel(input, exemplar_embeddings, log_exemplar_event_counts,
           log_exemplar_censor_counts, log_baseline_event_counts,
           log_baseline_censor_counts):
    sq, labels = _knn(input, exemplar_embeddings)
    idx = labels.reshape(-1)
    gev, gce = _sc_gather(log_exemplar_event_counts,
                          log_exemplar_censor_counts, idx)
    lbe = log_baseline_event_counts.reshape(1, NT)
    lbc = log_baseline_censor_counts.reshape(1, NT)
    return _combine(sq, gev, gce, lbe, lbc)


# speedup vs baseline: 1.1112x; 1.0333x over previous
import functools

import jax
import jax.numpy as jnp
from jax.experimental import pallas as pl
from jax.experimental.pallas import tpu as pltpu

KNB = 32
TAU2 = 4.0
NQ = 1024
NE = 100000
ED = 32
NT = 32


QB = 128


def _combine_body(sq_ref, gev_ref, gce_ref, lbe_ref, lbc_ref, out_ref):
    sq = sq_ref[...]
    w = jnp.exp(-sq) * (sq <= TAU2).astype(jnp.float32)
    ia = jax.lax.broadcasted_iota(jnp.int32, (NT, NT), 0)
    ib = jax.lax.broadcasted_iota(jnp.int32, (NT, NT), 1)
    ut = (ia >= ib).astype(jnp.float32)
    ev = jnp.exp(gev_ref[...])
    ar = ev + jnp.exp(gce_ref[...])
    risk = jnp.dot(ar, ut, preferred_element_type=jnp.float32,
                   precision=jax.lax.Precision.HIGHEST)
    w3 = w[:, :, None]
    nm = jnp.sum(ev.reshape(QB, KNB, NT) * w3, axis=1)
    dn = jnp.sum(risk.reshape(QB, KNB, NT) * w3, axis=1)
    bev = jnp.exp(lbe_ref[...])
    bar = jnp.dot(bev + jnp.exp(lbc_ref[...]), ut,
                  preferred_element_type=jnp.float32,
                  precision=jax.lax.Precision.HIGHEST)
    numer = nm + bev
    denom = dn + bar + 1e-12
    out_ref[...] = jnp.clip(numer / denom, 1e-12, 1.0 - 1e-12)


@functools.partial(jax.jit)
def _combine(sq, gev, gce, lbe, lbc):
    return pl.pallas_call(
        _combine_body,
        grid=(NQ // QB,),
        in_specs=[
            pl.BlockSpec((QB, KNB), lambda i: (i, 0)),
            pl.BlockSpec((QB * KNB, NT), lambda i: (i, 0)),
            pl.BlockSpec((QB * KNB, NT), lambda i: (i, 0)),
            pl.BlockSpec((1, NT), lambda i: (0, 0)),
            pl.BlockSpec((1, NT), lambda i: (0, 0)),
        ],
        out_specs=pl.BlockSpec((QB, NT), lambda i: (i, 0)),
        out_shape=jax.ShapeDtypeStruct((NQ, NT), jnp.float32),
    )(sq, gev, gce, lbe, lbc)


def _sc_gather_body(ev_hbm, ce_hbm, idx_hbm, oev_hbm, oce_hbm,
                    idx_v, rows_ev, rows_ce, sem):
    wid = jax.lax.axis_index("s") * 2 + jax.lax.axis_index("c")
    bpw = idx_v.shape[0]
    base = wid * bpw
    pltpu.sync_copy(idx_hbm.at[pl.ds(base, bpw)], idx_v)
    a = pltpu.async_copy(ev_hbm.at[idx_v], rows_ev, sem)
    b = pltpu.async_copy(ce_hbm.at[idx_v], rows_ce, sem)
    a.wait()
    b.wait()
    pltpu.sync_copy(rows_ev, oev_hbm.at[pl.ds(base, bpw)])
    pltpu.sync_copy(rows_ce, oce_hbm.at[pl.ds(base, bpw)])


def _sc_gather(ev, ce, idx):
    from jax.experimental.pallas import tpu_sc as plsc
    b = idx.shape[0]
    nt = ev.shape[1]
    nw = 32
    bpw = b // nw
    mesh = plsc.VectorSubcoreMesh(core_axis_name="c", subcore_axis_name="s")
    f = pl.kernel(
        _sc_gather_body,
        mesh=mesh,
        compiler_params=pltpu.CompilerParams(use_tc_tiling_on_sc=False),
        out_type=[
            jax.ShapeDtypeStruct((b, nt), jnp.float32),
            jax.ShapeDtypeStruct((b, nt), jnp.float32),
        ],
        scratch_types=[
            pltpu.VMEM((bpw,), jnp.int32),
            pltpu.VMEM((bpw, nt), jnp.float32),
            pltpu.VMEM((bpw, nt), jnp.float32),
            pltpu.SemaphoreType.DMA,
        ],
    )
    return f(ev, ce, idx)


_F32_INF = float("inf")
_I32_BIG = 2**31 - 1


def _rowmin_arg(x, idx):
    m = jnp.min(x, axis=1, keepdims=True)
    cand = jnp.where(x == m, idx, _I32_BIG)
    return m, jnp.min(cand, axis=1, keepdims=True)


def _knn_body(nb, eb, q_ref, e_ref, e2_ref, tv_ref, ti_ref,
              bv_ref, bi_ref):
    i = pl.program_id(0)
    j = jax.lax.rem(i, nb)
    nq = q_ref.shape[0]
    q = q_ref[...]
    e = e_ref[...]
    q2 = jnp.sum(q * q, axis=1, keepdims=True)
    mm = jax.lax.dot_general(q, e, (((1,), (1,)), ((), ())),
                             preferred_element_type=jnp.float32)
    d2 = jnp.maximum((q2 + e2_ref[...]) - 2.0 * mm, 0.0)
    lane = jax.lax.broadcasted_iota(jnp.int32, (nq, eb), 1)
    gidx = j * eb + lane

    @pl.when(i == 0)
    def _init():
        bv_ref[...] = jnp.full((nq, eb), _F32_INF, jnp.float32)
        bi_ref[...] = jnp.zeros((nq, eb), jnp.int32)

    @pl.when(i < nb)
    def _phase1():
        bv = bv_ref[...]
        sel = d2 < bv
        bv_ref[...] = jnp.where(sel, d2, bv)
        bi_ref[...] = jnp.where(sel, gidx, bi_ref[...])

    @pl.when(i == nb)
    def _extract():
        bv = bv_ref[...]
        bi = bi_ref[...]
        vals, idxs = [], []
        for _ in range(KNB):
            m, am = _rowmin_arg(bv, lane)
            vals.append(m)
            hit = lane == am
            idxs.append(jnp.min(jnp.where(hit, bi, _I32_BIG), axis=1,
                                keepdims=True))
            bv = jnp.where(hit, _F32_INF, bv)
        tv_ref[...] = jnp.concatenate(vals, axis=1)
        ti_ref[...] = jnp.concatenate(idxs, axis=1)

    @pl.when(i >= nb)
    def _phase2():
        d2m = jnp.where(gidx == bi_ref[...], _F32_INF, d2)
        tv = tv_ref[...]
        ti = ti_ref[...]
        t = jnp.max(tv, axis=1, keepdims=True)
        lane32 = jax.lax.broadcasted_iota(jnp.int32, (nq, KNB), 1)
        m0, gm0 = _rowmin_arg(d2m, gidx)

        def cond(c):
            m, gm, tv, ti, t = c
            return jnp.any(m < t)

        def body(c):
            m, gm, tv, ti, t = c
            ins = m < t
            pos = jnp.min(jnp.where(tv == t, lane32, _I32_BIG), axis=1,
                          keepdims=True)
            hit = ins & (lane32 == pos)
            tv = jnp.where(hit, m, tv)
            ti = jnp.where(hit, gm, ti)
            t = jnp.max(tv, axis=1, keepdims=True)
            act = (d2m > m) | ((d2m == m) & (gidx > gm))
            dd = jnp.where(act, d2m, _F32_INF)
            m2, gm2 = _rowmin_arg(dd, gidx)
            return m2, gm2, tv, ti, t

        _, _, tv, ti, _ = jax.lax.while_loop(cond, body, (m0, gm0, tv, ti, t))
        tv_ref[...] = tv
        ti_ref[...] = ti


def _knn(q, emb, eb=1024):
    nq, ed = q.shape
    ne = emb.shape[0]
    nb = (ne + eb - 1) // eb
    pad = nb * eb - ne
    e2 = jnp.sum(emb * emb, axis=1)
    if pad:
        emb = jnp.concatenate([emb, jnp.zeros((pad, ed), jnp.float32)], axis=0)
        e2 = jnp.concatenate([e2, jnp.full((pad,), 1e9, jnp.float32)], axis=0)
    e2 = e2.reshape(1, nb * eb)
    body = functools.partial(_knn_body, nb, eb)
    return pl.pallas_call(
        body,
        grid=(2 * nb,),
        in_specs=[
            pl.BlockSpec((nq, ed), lambda i: (0, 0)),
            pl.BlockSpec((eb, ed), lambda i: (jax.lax.rem(i, nb), 0)),
            pl.BlockSpec((1, eb), lambda i: (0, jax.lax.rem(i, nb))),
        ],
        out_specs=[
            pl.BlockSpec((nq, KNB), lambda i: (0, 0)),
            pl.BlockSpec((nq, KNB), lambda i: (0, 0)),
        ],
        out_shape=[
            jax.ShapeDtypeStruct((nq, KNB), jnp.float32),
            jax.ShapeDtypeStruct((nq, KNB), jnp.int32),
        ],
        scratch_shapes=[
            pltpu.VMEM((nq, eb), jnp.float32),
            pltpu.VMEM((nq, eb), jnp.int32),
        ],
    )(q, emb, e2)


def kernel(input, exemplar_embeddings, log_exemplar_event_counts,
           log_exemplar_censor_counts, log_baseline_event_counts,
           log_baseline_censor_counts):
    sq, labels = _knn(input, exemplar_embeddings)
    idx = labels.reshape(-1)
    gev, gce = _sc_gather(log_exemplar_event_counts,
                          log_exemplar_censor_counts, idx)
    lbe = log_baseline_event_counts.reshape(1, NT)
    lbc = log_baseline_censor_counts.reshape(1, NT)
    return _combine(sq, gev, gce, lbe, lbc)
